# Initial kernel scaffold; baseline (speedup 1.0000x reference)
import jax
import jax.numpy as jnp
from jax.experimental import pallas as pl


def kernel(x, edge_index, x0, W_in, W_skip, W_conv, b_conv, W_fc, b_fc):
    raise NotImplementedError("write your pallas kernel here")


# speedup vs baseline: 12.7682x; 12.7682x over previous
import functools

import jax
import jax.numpy as jnp
from jax import lax
from jax.experimental import pallas as pl
from jax.experimental.pallas import tpu as pltpu
from jax.experimental.pallas import tpu_sc as plsc

N = 10000
NP = 10240
E = 160000
EP = 163840
NW = 32
NC = 2
NS = 16
CH = 128
NCH = EP // NW // CH
ERODS = EP // CH
YW = 128
HID = 64
RT = 1024
NRT = NP // RT
MT = 2048
KT = 1024


def _prep_body(x_ref, x0_ref, win_ref, wskip_ref, wc0_ref,
               xh_ref, xskip_ref, xw1_ref):
    xh = jnp.dot(x_ref[...], win_ref[...], preferred_element_type=jnp.float32)
    x0h = jnp.dot(x0_ref[...], win_ref[...], preferred_element_type=jnp.float32)
    xh_ref[...] = xh
    xskip_ref[...] = jnp.dot(x0h, wskip_ref[...],
                             preferred_element_type=jnp.float32)
    xw1 = jnp.dot(xh, wc0_ref[...], preferred_element_type=jnp.float32)
    xw1_ref[...] = jnp.concatenate(
        [xw1, jnp.zeros((RT, YW - HID), jnp.float32)], axis=1)


def _prep(xp, x0p, w_in, w_skip, wc0):
    inc = xp.shape[1]
    return pl.pallas_call(
        _prep_body,
        grid=(NRT,),
        in_specs=[
            pl.BlockSpec((RT, inc), lambda i: (i, 0)),
            pl.BlockSpec((RT, inc), lambda i: (i, 0)),
            pl.BlockSpec((inc, HID), lambda i: (0, 0)),
            pl.BlockSpec((HID, HID), lambda i: (0, 0)),
            pl.BlockSpec((HID, HID), lambda i: (0, 0)),
        ],
        out_specs=[
            pl.BlockSpec((RT, HID), lambda i: (i, 0)),
            pl.BlockSpec((RT, HID), lambda i: (i, 0)),
            pl.BlockSpec((RT, YW), lambda i: (i, 0)),
        ],
        out_shape=[
            jax.ShapeDtypeStruct((NP, HID), jnp.float32),
            jax.ShapeDtypeStruct((NP, HID), jnp.float32),
            jax.ShapeDtypeStruct((NP, YW), jnp.float32),
        ],
    )(xp, x0p, w_in, w_skip, wc0)


def _flat_body(src_ref, dst_ref, out_ref):
    out_ref[...] = src_ref[...] * NP + dst_ref[...]


def _flat(src2, dst2):
    return pl.pallas_call(
        _flat_body,
        out_shape=jax.ShapeDtypeStruct((ERODS, CH), jnp.int32),
    )(src2, dst2)


def _build_a_kernel():
    mesh = plsc.VectorSubcoreMesh(core_axis_name="c", subcore_axis_name="s")

    @functools.partial(
        pl.kernel,
        out_type=(),
        mesh=mesh,
        scratch_types=[
            pltpu.VMEM((NCH, CH), jnp.int32),
            pltpu.VMEM((CH,), jnp.float32),
        ],
    )
    def build_a(flat_hbm, a_ref, idx_v, ones_v):
        c = lax.axis_index("c")
        s = lax.axis_index("s")
        wid = s * NC + c
        for i in range(CH // 16):
            ones_v[pl.ds(i * 16, 16)] = jnp.ones((16,), jnp.float32)
        pltpu.sync_copy(flat_hbm.at[pl.ds(wid * NCH, NCH)], idx_v)

        def scat(j, carry):
            pltpu.sync_copy(ones_v, a_ref.at[idx_v.at[j]])
            return carry

        lax.fori_loop(0, NCH, scat, 0)

    return build_a


def _conv_body(a_ref, o_ref):
    o_ref[...] = a_ref[...].astype(jnp.int8)


def _conv_int8(a2d):
    blk = 256
    return pl.pallas_call(
        _conv_body,
        grid=(NP // blk,),
        in_specs=[pl.BlockSpec((blk, NP), lambda i: (i, 0))],
        out_specs=pl.BlockSpec((blk, NP), lambda i: (i, 0)),
        out_shape=jax.ShapeDtypeStruct((NP, NP), jnp.int8),
    )(a2d)


def _a2_body(a_ref, b_ref, s_ref, acc_ref):
    k = pl.program_id(2)
    part = jnp.dot(a_ref[...], b_ref[...], preferred_element_type=jnp.int32)

    @pl.when(k == 0)
    def _():
        acc_ref[...] = part

    @pl.when(k > 0)
    def _():
        acc_ref[...] += part

    @pl.when(k == (NP // KT) - 1)
    def _():
        i = pl.program_id(0)
        j = pl.program_id(1)
        ri = i * MT + lax.broadcasted_iota(jnp.int32, (MT, MT), 0)
        ci = j * MT + lax.broadcasted_iota(jnp.int32, (MT, MT), 1)
        s_ref[...] = ((acc_ref[...] > 0) & (ri != ci)).astype(jnp.int8)


def _a2(a8):
    g = NP // MT
    return pl.pallas_call(
        _a2_body,
        grid=(g, g, NP // KT),
        in_specs=[
            pl.BlockSpec((MT, KT), lambda i, j, k: (i, k)),
            pl.BlockSpec((KT, MT), lambda i, j, k: (k, j)),
        ],
        out_specs=pl.BlockSpec((MT, MT), lambda i, j, k: (i, j)),
        out_shape=jax.ShapeDtypeStruct((NP, NP), jnp.int8),
        scratch_shapes=[pltpu.VMEM((MT, MT), jnp.int32)],
    )(a8, a8)


def _segsum_kernel(W):
    mesh = plsc.VectorSubcoreMesh(core_axis_name="c", subcore_axis_name="s")
    rows_per_tile = NP // NS
    zrows = 8

    @functools.partial(
        pl.kernel,
        out_type=jax.ShapeDtypeStruct((NC, NP, W), jnp.float32),
        mesh=mesh,
        scratch_types=[
            pltpu.VMEM((NCH, CH), jnp.int32),
            pltpu.VMEM((NCH, CH), jnp.int32),
            pltpu.VMEM((CH, W), jnp.float32),
            pltpu.VMEM((zrows, W), jnp.float32),
            pltpu.VMEM_SHARED((NP, W), jnp.float32),
            pltpu.SemaphoreType.DMA,
        ],
    )
    def seg(tbl_hbm, gidx_hbm, sidx_hbm, out_hbm,
            gidx, sidx, buf, zbuf, acc, sem):
        c = lax.axis_index("c")
        s = lax.axis_index("s")
        wid = s * NC + c
        for r in range(zrows):
            for col in range(W // 16):
                zbuf[r, pl.ds(col * 16, 16)] = jnp.zeros((16,), jnp.float32)
        row0 = s * rows_per_tile

        def zf(i, carry):
            pltpu.sync_copy(zbuf, acc.at[pl.ds(row0 + i * zrows, zrows)])
            return carry

        lax.fori_loop(0, rows_per_tile // zrows, zf, 0)
        pltpu.sync_copy(gidx_hbm.at[pl.ds(wid * NCH, NCH)], gidx)
        pltpu.sync_copy(sidx_hbm.at[pl.ds(wid * NCH, NCH)], sidx)
        plsc.subcore_barrier()

        def step(j, carry):
            pltpu.async_copy(tbl_hbm.at[gidx.at[j]], buf, sem).wait()
            pltpu.sync_copy(buf, acc.at[sidx.at[j]], add=True)
            return carry

        lax.fori_loop(0, NCH, step, 0)
        plsc.subcore_barrier()

        def wo(i, carry):
            r = row0 + i * zrows * 8
            pltpu.sync_copy(acc.at[pl.ds(r, zrows * 8)],
                            out_hbm.at[c, pl.ds(r, zrows * 8)])
            return carry

        lax.fori_loop(0, rows_per_tile // (zrows * 8), wo, 0)

    return seg


def _mid_body(agg_ref, b_ref, y_ref):
    i = pl.program_id(0)
    a = agg_ref[0][:, :HID] + agg_ref[1][:, :HID] + b_ref[...]
    x_agg = jnp.maximum(a, 0.0)
    rid = i * RT + lax.broadcasted_iota(jnp.int32, (RT, 1), 0)
    real = (rid < N).astype(jnp.float32)
    x_agg = x_agg * real
    sqn = jnp.sum(x_agg * x_agg, axis=1, keepdims=True)
    y_ref[...] = jnp.concatenate(
        [x_agg, sqn, real, jnp.zeros((RT, YW - HID - 2), jnp.float32)], axis=1)


def _mid(agg, b):
    return pl.pallas_call(
        _mid_body,
        grid=(NRT,),
        in_specs=[
            pl.BlockSpec((NC, RT, YW), lambda i: (0, i, 0)),
            pl.BlockSpec((1, HID), lambda i: (0, 0)),
        ],
        out_specs=pl.BlockSpec((RT, YW), lambda i: (i, 0)),
        out_shape=jax.ShapeDtypeStruct((NP, YW), jnp.float32),
    )(agg, b)


def _combine_body(last, s_ref, yk_ref, yi_ref, p_ref, x_ref, xsk_ref,
                  wn_ref, bn_ref, *rest):
    if last:
        (o_ref, acc_ref) = rest
    else:
        (o_ref, ow_ref, acc_ref) = rest
    k = pl.program_id(1)
    part = jnp.dot(s_ref[...].astype(jnp.float32), yk_ref[...],
                   preferred_element_type=jnp.float32)

    @pl.when(k == 0)
    def _():
        acc_ref[...] = part

    @pl.when(k > 0)
    def _():
        acc_ref[...] += part

    @pl.when(k == NRT - 1)
    def _():
        u = acc_ref[...]
        yi = yi_ref[...]
        x_agg = yi[:, :HID]
        sqn = yi[:, HID:HID + 1]
        pm = p_ref[0] + p_ref[1]
        num_s = (pm[:, HID + 1:HID + 2] * sqn + pm[:, HID:HID + 1]
                 - 2.0 * jnp.sum(x_agg * pm[:, :HID], axis=1, keepdims=True))
        g_s = jnp.tanh(num_s / (pm[:, HID + 1:HID + 2] + 1e-10))
        num_q = (u[:, HID + 1:HID + 2] * sqn + u[:, HID:HID + 1]
                 - 2.0 * jnp.sum(x_agg * u[:, :HID], axis=1, keepdims=True))
        g_q = jnp.tanh(num_q / (u[:, HID + 1:HID + 2] + 1e-10))
        denom = 1.0 + g_s + g_q
        x_new = (x_ref[...] + g_s * x_agg + g_q * xsk_ref[...]) / denom
        proj = jnp.dot(x_new, wn_ref[...],
                       preferred_element_type=jnp.float32) + bn_ref[...]
        if last:
            o_ref[...] = proj
        else:
            o_ref[...] = x_new
            ow_ref[...] = jnp.concatenate(
                [proj, jnp.zeros((RT, YW - HID), jnp.float32)], axis=1)


def _combine(last, s8, y, p, x, xsk, wn, bn):
    outs = [jax.ShapeDtypeStruct((NP, HID), jnp.float32)]
    ospecs = [pl.BlockSpec((RT, HID), lambda i, k: (i, 0))]
    if not last:
        outs.append(jax.ShapeDtypeStruct((NP, YW), jnp.float32))
        ospecs.append(pl.BlockSpec((RT, YW), lambda i, k: (i, 0)))
    res = pl.pallas_call(
        functools.partial(_combine_body, last),
        grid=(NRT, NRT),
        in_specs=[
            pl.BlockSpec((RT, RT), lambda i, k: (i, k)),
            pl.BlockSpec((RT, YW), lambda i, k: (k, 0)),
            pl.BlockSpec((RT, YW), lambda i, k: (i, 0)),
            pl.BlockSpec((NC, RT, YW), lambda i, k: (0, i, 0)),
            pl.BlockSpec((RT, HID), lambda i, k: (i, 0)),
            pl.BlockSpec((RT, HID), lambda i, k: (i, 0)),
            pl.BlockSpec((HID, HID), lambda i, k: (0, 0)),
            pl.BlockSpec((1, HID), lambda i, k: (0, 0)),
        ],
        out_specs=ospecs[0] if last else ospecs,
        out_shape=outs[0] if last else outs,
        scratch_shapes=[pltpu.VMEM((RT, YW), jnp.float32)],
    )(s8, y, y, p, x, xsk, wn, bn)
    return res


_BUILD_A = _build_a_kernel()
_SEG128 = _segsum_kernel(YW)


def kernel(x, edge_index, x0, W_in, W_skip, W_conv, b_conv, W_fc, b_fc):
    f32 = jnp.float32
    xp = jnp.pad(x.astype(f32), ((0, NP - N), (0, 0)))
    x0p = jnp.pad(x0.astype(f32), ((0, NP - N), (0, 0)))
    ei = edge_index.astype(jnp.int32)
    src2 = jnp.pad(ei[0], (0, EP - E), constant_values=NP - 1).reshape(ERODS, CH)
    dst2 = jnp.pad(ei[1], (0, EP - E), constant_values=NP - 1).reshape(ERODS, CH)

    xh, xskip, xw1 = _prep(xp, x0p, W_in.astype(f32), W_skip.astype(f32),
                           W_conv[0].astype(f32))
    flat = _flat(src2, dst2)

    a_ref = jax.new_ref(jnp.zeros((NP * NP,), f32))
    _BUILD_A(flat, a_ref)
    a8 = _conv_int8(a_ref[...].reshape(NP, NP))
    s8 = _a2(a8)

    b0 = b_conv[0].astype(f32).reshape(1, HID)
    b1 = b_conv[1].astype(f32).reshape(1, HID)
    zb = jnp.zeros((1, HID), f32)
    bfc = b_fc.astype(f32).reshape(1, HID)

    agg1 = _SEG128(xw1, src2, dst2)
    y1 = _mid(agg1, b0)
    p1 = _SEG128(y1, dst2, src2)
    x1, xw2 = _combine(False, s8, y1, p1, xh, xskip, W_conv[1].astype(f32), zb)

    agg2 = _SEG128(xw2, src2, dst2)
    y2 = _mid(agg2, b1)
    p2 = _SEG128(y2, dst2, src2)
    out = _combine(True, s8, y2, p2, x1, xskip, W_fc.astype(f32), bfc)
    return out[:N]


# speedup vs baseline: 12.8086x; 1.0032x over previous
import functools

import jax
import jax.numpy as jnp
from jax import lax
from jax.experimental import pallas as pl
from jax.experimental.pallas import tpu as pltpu
from jax.experimental.pallas import tpu_sc as plsc

N = 10000
NP = 10240
E = 160000
EP = 163840
NW = 32
NC = 2
NS = 16
CH = 128
NCH = EP // NW // CH
ERODS = EP // CH
YW = 128
HID = 64
RT = 1024
NRT = NP // RT
MT = 2048
KT = 1024


def _prep_body(x_ref, x0_ref, win_ref, wskip_ref, wc0_ref,
               xh_ref, xskip_ref, xw1_ref):
    xh = jnp.dot(x_ref[...], win_ref[...], preferred_element_type=jnp.float32)
    x0h = jnp.dot(x0_ref[...], win_ref[...], preferred_element_type=jnp.float32)
    xh_ref[...] = xh
    xskip_ref[...] = jnp.dot(x0h, wskip_ref[...],
                             preferred_element_type=jnp.float32)
    xw1 = jnp.dot(xh, wc0_ref[...], preferred_element_type=jnp.float32)
    xw1_ref[...] = jnp.concatenate(
        [xw1, jnp.zeros((RT, YW - HID), jnp.float32)], axis=1)


def _prep(xp, x0p, w_in, w_skip, wc0):
    inc = xp.shape[1]
    return pl.pallas_call(
        _prep_body,
        grid=(NRT,),
        in_specs=[
            pl.BlockSpec((RT, inc), lambda i: (i, 0)),
            pl.BlockSpec((RT, inc), lambda i: (i, 0)),
            pl.BlockSpec((inc, HID), lambda i: (0, 0)),
            pl.BlockSpec((HID, HID), lambda i: (0, 0)),
            pl.BlockSpec((HID, HID), lambda i: (0, 0)),
        ],
        out_specs=[
            pl.BlockSpec((RT, HID), lambda i: (i, 0)),
            pl.BlockSpec((RT, HID), lambda i: (i, 0)),
            pl.BlockSpec((RT, YW), lambda i: (i, 0)),
        ],
        out_shape=[
            jax.ShapeDtypeStruct((NP, HID), jnp.float32),
            jax.ShapeDtypeStruct((NP, HID), jnp.float32),
            jax.ShapeDtypeStruct((NP, YW), jnp.float32),
        ],
    )(xp, x0p, w_in, w_skip, wc0)


def _flat_body(src_ref, dst_ref, out_ref):
    out_ref[...] = src_ref[...] * NP + dst_ref[...]


def _flat(src2, dst2):
    return pl.pallas_call(
        _flat_body,
        out_shape=jax.ShapeDtypeStruct((ERODS, CH), jnp.int32),
    )(src2, dst2)


def _build_a_kernel():
    mesh = plsc.VectorSubcoreMesh(core_axis_name="c", subcore_axis_name="s")

    grp = 8

    @functools.partial(
        pl.kernel,
        out_type=(),
        mesh=mesh,
        scratch_types=[
            pltpu.VMEM((NCH, CH), jnp.int32),
            pltpu.VMEM((CH,), jnp.float32),
            pltpu.SemaphoreType.DMA,
        ],
    )
    def build_a(flat_hbm, a_ref, idx_v, ones_v, sem):
        c = lax.axis_index("c")
        s = lax.axis_index("s")
        wid = s * NC + c
        for i in range(CH // 16):
            ones_v[pl.ds(i * 16, 16)] = jnp.ones((16,), jnp.float32)
        pltpu.sync_copy(flat_hbm.at[pl.ds(wid * NCH, NCH)], idx_v)

        def scat(g, carry):
            descs = [pltpu.async_copy(ones_v, a_ref.at[idx_v.at[g * grp + u]],
                                      sem) for u in range(grp)]
            for d in descs:
                d.wait()
            return carry

        lax.fori_loop(0, NCH // grp, scat, 0)

    return build_a


def _conv_body(a_ref, o_ref):
    o_ref[...] = a_ref[...].astype(jnp.int8)


def _conv_int8(a2d):
    blk = 256
    return pl.pallas_call(
        _conv_body,
        grid=(NP // blk,),
        in_specs=[pl.BlockSpec((blk, NP), lambda i: (i, 0))],
        out_specs=pl.BlockSpec((blk, NP), lambda i: (i, 0)),
        out_shape=jax.ShapeDtypeStruct((NP, NP), jnp.int8),
    )(a2d)


def _a2_body(a_ref, b_ref, s_ref, acc_ref):
    k = pl.program_id(2)
    part = jnp.dot(a_ref[...], b_ref[...], preferred_element_type=jnp.int32)

    @pl.when(k == 0)
    def _():
        acc_ref[...] = part

    @pl.when(k > 0)
    def _():
        acc_ref[...] += part

    @pl.when(k == (NP // KT) - 1)
    def _():
        i = pl.program_id(0)
        j = pl.program_id(1)
        ri = i * MT + lax.broadcasted_iota(jnp.int32, (MT, MT), 0)
        ci = j * MT + lax.broadcasted_iota(jnp.int32, (MT, MT), 1)
        s_ref[...] = ((acc_ref[...] > 0) & (ri != ci)).astype(jnp.int8)


def _a2(a8):
    g = NP // MT
    return pl.pallas_call(
        _a2_body,
        grid=(g, g, NP // KT),
        in_specs=[
            pl.BlockSpec((MT, KT), lambda i, j, k: (i, k)),
            pl.BlockSpec((KT, MT), lambda i, j, k: (k, j)),
        ],
        out_specs=pl.BlockSpec((MT, MT), lambda i, j, k: (i, j)),
        out_shape=jax.ShapeDtypeStruct((NP, NP), jnp.int8),
        scratch_shapes=[pltpu.VMEM((MT, MT), jnp.int32)],
    )(a8, a8)


def _segsum_kernel(W):
    mesh = plsc.VectorSubcoreMesh(core_axis_name="c", subcore_axis_name="s")
    rows_per_tile = NP // NS
    zrows = 8
    nbuf = 2

    @functools.partial(
        pl.kernel,
        out_type=jax.ShapeDtypeStruct((NC, NP, W), jnp.float32),
        mesh=mesh,
        scratch_types=[
            pltpu.VMEM((NCH, CH), jnp.int32),
            pltpu.VMEM((NCH, CH), jnp.int32),
            [pltpu.VMEM((CH, W), jnp.float32) for _ in range(nbuf)],
            pltpu.VMEM((zrows, W), jnp.float32),
            pltpu.VMEM_SHARED((NP, W), jnp.float32),
            [pltpu.SemaphoreType.DMA for _ in range(nbuf)],
        ],
    )
    def seg(tbl_hbm, gidx_hbm, sidx_hbm, out_hbm,
            gidx, sidx, bufs, zbuf, acc, sems):
        c = lax.axis_index("c")
        s = lax.axis_index("s")
        wid = s * NC + c
        for r in range(zrows):
            for col in range(W // 16):
                zbuf[r, pl.ds(col * 16, 16)] = jnp.zeros((16,), jnp.float32)
        row0 = s * rows_per_tile

        def zf(i, carry):
            pltpu.sync_copy(zbuf, acc.at[pl.ds(row0 + i * zrows, zrows)])
            return carry

        lax.fori_loop(0, rows_per_tile // zrows, zf, 0)
        pltpu.sync_copy(gidx_hbm.at[pl.ds(wid * NCH, NCH)], gidx)
        pltpu.sync_copy(sidx_hbm.at[pl.ds(wid * NCH, NCH)], sidx)
        plsc.subcore_barrier()

        def step(g, carry):
            descs = [pltpu.async_copy(tbl_hbm.at[gidx.at[g * nbuf + u]],
                                      bufs[u], sems[u]) for u in range(nbuf)]
            for u in range(nbuf):
                descs[u].wait()
                pltpu.sync_copy(bufs[u], acc.at[sidx.at[g * nbuf + u]],
                                add=True)
            return carry

        lax.fori_loop(0, NCH // nbuf, step, 0)
        plsc.subcore_barrier()

        def wo(i, carry):
            r = row0 + i * zrows * 8
            pltpu.sync_copy(acc.at[pl.ds(r, zrows * 8)],
                            out_hbm.at[c, pl.ds(r, zrows * 8)])
            return carry

        lax.fori_loop(0, rows_per_tile // (zrows * 8), wo, 0)

    return seg


def _mid_body(agg_ref, b_ref, y_ref):
    i = pl.program_id(0)
    a = agg_ref[0][:, :HID] + agg_ref[1][:, :HID] + b_ref[...]
    x_agg = jnp.maximum(a, 0.0)
    rid = i * RT + lax.broadcasted_iota(jnp.int32, (RT, 1), 0)
    real = (rid < N).astype(jnp.float32)
    x_agg = x_agg * real
    sqn = jnp.sum(x_agg * x_agg, axis=1, keepdims=True)
    y_ref[...] = jnp.concatenate(
        [x_agg, sqn, real, jnp.zeros((RT, YW - HID - 2), jnp.float32)], axis=1)


def _mid(agg, b):
    return pl.pallas_call(
        _mid_body,
        grid=(NRT,),
        in_specs=[
            pl.BlockSpec((NC, RT, YW), lambda i: (0, i, 0)),
            pl.BlockSpec((1, HID), lambda i: (0, 0)),
        ],
        out_specs=pl.BlockSpec((RT, YW), lambda i: (i, 0)),
        out_shape=jax.ShapeDtypeStruct((NP, YW), jnp.float32),
    )(agg, b)


def _combine_body(last, s_ref, yk_ref, yi_ref, p_ref, x_ref, xsk_ref,
                  wn_ref, bn_ref, *rest):
    if last:
        (o_ref, acc_ref) = rest
    else:
        (o_ref, ow_ref, acc_ref) = rest
    k = pl.program_id(1)
    part = jnp.dot(s_ref[...].astype(jnp.float32), yk_ref[...],
                   preferred_element_type=jnp.float32)

    @pl.when(k == 0)
    def _():
        acc_ref[...] = part

    @pl.when(k > 0)
    def _():
        acc_ref[...] += part

    @pl.when(k == NRT - 1)
    def _():
        u = acc_ref[...]
        yi = yi_ref[...]
        x_agg = yi[:, :HID]
        sqn = yi[:, HID:HID + 1]
        pm = p_ref[0] + p_ref[1]
        num_s = (pm[:, HID + 1:HID + 2] * sqn + pm[:, HID:HID + 1]
                 - 2.0 * jnp.sum(x_agg * pm[:, :HID], axis=1, keepdims=True))
        g_s = jnp.tanh(num_s / (pm[:, HID + 1:HID + 2] + 1e-10))
        num_q = (u[:, HID + 1:HID + 2] * sqn + u[:, HID:HID + 1]
                 - 2.0 * jnp.sum(x_agg * u[:, :HID], axis=1, keepdims=True))
        g_q = jnp.tanh(num_q / (u[:, HID + 1:HID + 2] + 1e-10))
        denom = 1.0 + g_s + g_q
        x_new = (x_ref[...] + g_s * x_agg + g_q * xsk_ref[...]) / denom
        proj = jnp.dot(x_new, wn_ref[...],
                       preferred_element_type=jnp.float32) + bn_ref[...]
        if last:
            o_ref[...] = proj
        else:
            o_ref[...] = x_new
            ow_ref[...] = jnp.concatenate(
                [proj, jnp.zeros((RT, YW - HID), jnp.float32)], axis=1)


def _combine(last, s8, y, p, x, xsk, wn, bn):
    outs = [jax.ShapeDtypeStruct((NP, HID), jnp.float32)]
    ospecs = [pl.BlockSpec((RT, HID), lambda i, k: (i, 0))]
    if not last:
        outs.append(jax.ShapeDtypeStruct((NP, YW), jnp.float32))
        ospecs.append(pl.BlockSpec((RT, YW), lambda i, k: (i, 0)))
    res = pl.pallas_call(
        functools.partial(_combine_body, last),
        grid=(NRT, NRT),
        in_specs=[
            pl.BlockSpec((RT, RT), lambda i, k: (i, k)),
            pl.BlockSpec((RT, YW), lambda i, k: (k, 0)),
            pl.BlockSpec((RT, YW), lambda i, k: (i, 0)),
            pl.BlockSpec((NC, RT, YW), lambda i, k: (0, i, 0)),
            pl.BlockSpec((RT, HID), lambda i, k: (i, 0)),
            pl.BlockSpec((RT, HID), lambda i, k: (i, 0)),
            pl.BlockSpec((HID, HID), lambda i, k: (0, 0)),
            pl.BlockSpec((1, HID), lambda i, k: (0, 0)),
        ],
        out_specs=ospecs[0] if last else ospecs,
        out_shape=outs[0] if last else outs,
        scratch_shapes=[pltpu.VMEM((RT, YW), jnp.float32)],
    )(s8, y, y, p, x, xsk, wn, bn)
    return res


_BUILD_A = _build_a_kernel()
_SEG128 = _segsum_kernel(YW)


def kernel(x, edge_index, x0, W_in, W_skip, W_conv, b_conv, W_fc, b_fc):
    f32 = jnp.float32
    xp = jnp.pad(x.astype(f32), ((0, NP - N), (0, 0)))
    x0p = jnp.pad(x0.astype(f32), ((0, NP - N), (0, 0)))
    ei = edge_index.astype(jnp.int32)
    src2 = jnp.pad(ei[0], (0, EP - E), constant_values=NP - 1).reshape(ERODS, CH)
    dst2 = jnp.pad(ei[1], (0, EP - E), constant_values=NP - 1).reshape(ERODS, CH)

    xh, xskip, xw1 = _prep(xp, x0p, W_in.astype(f32), W_skip.astype(f32),
                           W_conv[0].astype(f32))
    flat = _flat(src2, dst2)

    a_ref = jax.new_ref(jnp.zeros((NP * NP,), f32))
    _BUILD_A(flat, a_ref)
    a8 = _conv_int8(a_ref[...].reshape(NP, NP))

    b0 = b_conv[0].astype(f32).reshape(1, HID)
    b1 = b_conv[1].astype(f32).reshape(1, HID)
    zb = jnp.zeros((1, HID), f32)
    bfc = b_fc.astype(f32).reshape(1, HID)

    s8 = _a2(a8)
    agg1 = _SEG128(xw1, src2, dst2)
    y1 = _mid(agg1, b0)
    p1 = _SEG128(y1, dst2, src2)
    x1, xw2 = _combine(False, s8, y1, p1, xh, xskip, W_conv[1].astype(f32), zb)

    agg2 = _SEG128(xw2, src2, dst2)
    y2 = _mid(agg2, b1)
    p2 = _SEG128(y2, dst2, src2)
    out = _combine(True, s8, y2, p2, x1, xskip, W_fc.astype(f32), bfc)
    return out[:N]


# speedup vs baseline: 17.9658x; 1.4026x over previous
import functools

import jax
import jax.numpy as jnp
from jax import lax
from jax.experimental import pallas as pl
from jax.experimental.pallas import tpu as pltpu
from jax.experimental.pallas import tpu_sc as plsc

N = 10000
NP = 10240
E = 160000
EP = 163840
NW = 32
NC = 2
NS = 16
CH = 128
NCH = EP // NW // CH
ERODS = EP // CH
YW = 128
HID = 64
RT = 1024
NRT = NP // RT
MT = 1024


def _prep_body(x_ref, x0_ref, win_ref, wskip_ref, wc0_ref,
               xh_ref, xskip_ref, xw1_ref):
    xh = jnp.dot(x_ref[...], win_ref[...], preferred_element_type=jnp.float32)
    x0h = jnp.dot(x0_ref[...], win_ref[...], preferred_element_type=jnp.float32)
    xh_ref[...] = xh
    xskip_ref[...] = jnp.dot(x0h, wskip_ref[...],
                             preferred_element_type=jnp.float32)
    xw1 = jnp.dot(xh, wc0_ref[...], preferred_element_type=jnp.float32)
    xw1_ref[...] = jnp.concatenate(
        [xw1, jnp.zeros((RT, YW - HID), jnp.float32)], axis=1)


def _prep(xp, x0p, w_in, w_skip, wc0):
    inc = xp.shape[1]
    return pl.pallas_call(
        _prep_body,
        grid=(NRT,),
        in_specs=[
            pl.BlockSpec((RT, inc), lambda i: (i, 0)),
            pl.BlockSpec((RT, inc), lambda i: (i, 0)),
            pl.BlockSpec((inc, HID), lambda i: (0, 0)),
            pl.BlockSpec((HID, HID), lambda i: (0, 0)),
            pl.BlockSpec((HID, HID), lambda i: (0, 0)),
        ],
        out_specs=[
            pl.BlockSpec((RT, HID), lambda i: (i, 0)),
            pl.BlockSpec((RT, HID), lambda i: (i, 0)),
            pl.BlockSpec((RT, YW), lambda i: (i, 0)),
        ],
        out_shape=[
            jax.ShapeDtypeStruct((NP, HID), jnp.float32),
            jax.ShapeDtypeStruct((NP, HID), jnp.float32),
            jax.ShapeDtypeStruct((NP, YW), jnp.float32),
        ],
    )(xp, x0p, w_in, w_skip, wc0)


def _flat_body(src_ref, dst_ref, out_ref):
    out_ref[...] = src_ref[...] * NP + dst_ref[...]


def _flat(src2, dst2):
    return pl.pallas_call(
        _flat_body,
        out_shape=jax.ShapeDtypeStruct((ERODS, CH), jnp.int32),
    )(src2, dst2)


def _build_a_kernel():
    mesh = plsc.VectorSubcoreMesh(core_axis_name="c", subcore_axis_name="s")

    grp = 8

    @functools.partial(
        pl.kernel,
        out_type=(),
        mesh=mesh,
        scratch_types=[
            pltpu.VMEM((NCH, CH), jnp.int32),
            pltpu.VMEM((CH,), jnp.float32),
            pltpu.SemaphoreType.DMA,
        ],
    )
    def build_a(flat_hbm, a_ref, idx_v, ones_v, sem):
        c = lax.axis_index("c")
        s = lax.axis_index("s")
        wid = s * NC + c
        for i in range(CH // 16):
            ones_v[pl.ds(i * 16, 16)] = jnp.ones((16,), jnp.float32)
        pltpu.sync_copy(flat_hbm.at[pl.ds(wid * NCH, NCH)], idx_v)

        def scat(g, carry):
            descs = [pltpu.async_copy(ones_v, a_ref.at[idx_v.at[g * grp + u]],
                                      sem) for u in range(grp)]
            for d in descs:
                d.wait()
            return carry

        lax.fori_loop(0, NCH // grp, scat, 0)

    return build_a


def _conv_body(a_ref, o_ref):
    o_ref[...] = a_ref[...].astype(jnp.int4)


def _conv_int8(a2d):
    blk = 256
    return pl.pallas_call(
        _conv_body,
        grid=(NP // blk,),
        in_specs=[pl.BlockSpec((blk, NP), lambda i: (i, 0))],
        out_specs=pl.BlockSpec((blk, NP), lambda i: (i, 0)),
        out_shape=jax.ShapeDtypeStruct((NP, NP), jnp.int4),
    )(a2d)


def _a2_body(a_ref, b_ref, s_ref):
    i = pl.program_id(0)
    j = pl.program_id(1)
    cnt = jnp.dot(a_ref[...], b_ref[...], preferred_element_type=jnp.int32)
    ri = i * MT + lax.broadcasted_iota(jnp.int32, (MT, MT), 0)
    ci = j * MT + lax.broadcasted_iota(jnp.int32, (MT, MT), 1)
    s_ref[...] = ((cnt > 0) & (ri != ci)).astype(jnp.bfloat16)


def _a2(a8):
    g = NP // MT
    return pl.pallas_call(
        _a2_body,
        grid=(g, g),
        in_specs=[
            pl.BlockSpec((MT, NP), lambda i, j: (i, 0)),
            pl.BlockSpec((NP, MT), lambda i, j: (0, j)),
        ],
        out_specs=pl.BlockSpec((MT, MT), lambda i, j: (i, j)),
        out_shape=jax.ShapeDtypeStruct((NP, NP), jnp.bfloat16),
    )(a8, a8)


def _segsum_kernel(W):
    mesh = plsc.VectorSubcoreMesh(core_axis_name="c", subcore_axis_name="s")
    rows_per_tile = NP // NS
    zrows = 8
    nbuf = 2

    @functools.partial(
        pl.kernel,
        out_type=jax.ShapeDtypeStruct((NC, NP, W), jnp.float32),
        mesh=mesh,
        scratch_types=[
            pltpu.VMEM((NCH, CH), jnp.int32),
            pltpu.VMEM((NCH, CH), jnp.int32),
            [pltpu.VMEM((CH, W), jnp.float32) for _ in range(nbuf)],
            pltpu.VMEM((zrows, W), jnp.float32),
            pltpu.VMEM_SHARED((NP, W), jnp.float32),
            [pltpu.SemaphoreType.DMA for _ in range(nbuf)],
        ],
    )
    def seg(tbl_hbm, gidx_hbm, sidx_hbm, out_hbm,
            gidx, sidx, bufs, zbuf, acc, sems):
        c = lax.axis_index("c")
        s = lax.axis_index("s")
        wid = s * NC + c
        for r in range(zrows):
            for col in range(W // 16):
                zbuf[r, pl.ds(col * 16, 16)] = jnp.zeros((16,), jnp.float32)
        row0 = s * rows_per_tile

        def zf(i, carry):
            pltpu.sync_copy(zbuf, acc.at[pl.ds(row0 + i * zrows, zrows)])
            return carry

        lax.fori_loop(0, rows_per_tile // zrows, zf, 0)
        pltpu.sync_copy(gidx_hbm.at[pl.ds(wid * NCH, NCH)], gidx)
        pltpu.sync_copy(sidx_hbm.at[pl.ds(wid * NCH, NCH)], sidx)
        plsc.subcore_barrier()

        def step(g, carry):
            descs = [pltpu.async_copy(tbl_hbm.at[gidx.at[g * nbuf + u]],
                                      bufs[u], sems[u]) for u in range(nbuf)]
            for u in range(nbuf):
                descs[u].wait()
                pltpu.sync_copy(bufs[u], acc.at[sidx.at[g * nbuf + u]],
                                add=True)
            return carry

        lax.fori_loop(0, NCH // nbuf, step, 0)
        plsc.subcore_barrier()

        def wo(i, carry):
            r = row0 + i * zrows * 8
            pltpu.sync_copy(acc.at[pl.ds(r, zrows * 8)],
                            out_hbm.at[c, pl.ds(r, zrows * 8)])
            return carry

        lax.fori_loop(0, rows_per_tile // (zrows * 8), wo, 0)

    return seg


def _mid_body(agg_ref, b_ref, y_ref):
    i = pl.program_id(0)
    a = agg_ref[0][:, :HID] + agg_ref[1][:, :HID] + b_ref[...]
    x_agg = jnp.maximum(a, 0.0)
    rid = i * RT + lax.broadcasted_iota(jnp.int32, (RT, 1), 0)
    real = (rid < N).astype(jnp.float32)
    x_agg = x_agg * real
    sqn = jnp.sum(x_agg * x_agg, axis=1, keepdims=True)
    y_ref[...] = jnp.concatenate(
        [x_agg, sqn, real, jnp.zeros((RT, YW - HID - 2), jnp.float32)], axis=1)


def _mid(agg, b):
    return pl.pallas_call(
        _mid_body,
        grid=(NRT,),
        in_specs=[
            pl.BlockSpec((NC, RT, YW), lambda i: (0, i, 0)),
            pl.BlockSpec((1, HID), lambda i: (0, 0)),
        ],
        out_specs=pl.BlockSpec((RT, YW), lambda i: (i, 0)),
        out_shape=jax.ShapeDtypeStruct((NP, YW), jnp.float32),
    )(agg, b)


def _combine_body(last, s_ref, yk_ref, yi_ref, p_ref, x_ref, xsk_ref,
                  wn_ref, bn_ref, *rest):
    if last:
        (o_ref, acc_ref) = rest
    else:
        (o_ref, ow_ref, acc_ref) = rest
    k = pl.program_id(1)
    yk = yk_ref[...]
    yhi = yk.astype(jnp.bfloat16)
    ylo = (yk - yhi.astype(jnp.float32)).astype(jnp.bfloat16)
    sb = s_ref[...]
    part = (jnp.dot(sb, yhi, preferred_element_type=jnp.float32)
            + jnp.dot(sb, ylo, preferred_element_type=jnp.float32))

    @pl.when(k == 0)
    def _():
        acc_ref[...] = part

    @pl.when(k > 0)
    def _():
        acc_ref[...] += part

    @pl.when(k == NRT - 1)
    def _():
        u = acc_ref[...]
        yi = yi_ref[...]
        x_agg = yi[:, :HID]
        sqn = yi[:, HID:HID + 1]
        pm = p_ref[0] + p_ref[1]
        num_s = (pm[:, HID + 1:HID + 2] * sqn + pm[:, HID:HID + 1]
                 - 2.0 * jnp.sum(x_agg * pm[:, :HID], axis=1, keepdims=True))
        g_s = jnp.tanh(num_s / (pm[:, HID + 1:HID + 2] + 1e-10))
        num_q = (u[:, HID + 1:HID + 2] * sqn + u[:, HID:HID + 1]
                 - 2.0 * jnp.sum(x_agg * u[:, :HID], axis=1, keepdims=True))
        g_q = jnp.tanh(num_q / (u[:, HID + 1:HID + 2] + 1e-10))
        denom = 1.0 + g_s + g_q
        x_new = (x_ref[...] + g_s * x_agg + g_q * xsk_ref[...]) / denom
        proj = jnp.dot(x_new, wn_ref[...],
                       preferred_element_type=jnp.float32) + bn_ref[...]
        if last:
            o_ref[...] = proj
        else:
            o_ref[...] = x_new
            ow_ref[...] = jnp.concatenate(
                [proj, jnp.zeros((RT, YW - HID), jnp.float32)], axis=1)


def _combine(last, s8, y, p, x, xsk, wn, bn):
    outs = [jax.ShapeDtypeStruct((NP, HID), jnp.float32)]
    ospecs = [pl.BlockSpec((RT, HID), lambda i, k: (i, 0))]
    if not last:
        outs.append(jax.ShapeDtypeStruct((NP, YW), jnp.float32))
        ospecs.append(pl.BlockSpec((RT, YW), lambda i, k: (i, 0)))
    res = pl.pallas_call(
        functools.partial(_combine_body, last),
        grid=(NRT, NRT),
        in_specs=[
            pl.BlockSpec((RT, RT), lambda i, k: (i, k)),
            pl.BlockSpec((RT, YW), lambda i, k: (k, 0)),
            pl.BlockSpec((RT, YW), lambda i, k: (i, 0)),
            pl.BlockSpec((NC, RT, YW), lambda i, k: (0, i, 0)),
            pl.BlockSpec((RT, HID), lambda i, k: (i, 0)),
            pl.BlockSpec((RT, HID), lambda i, k: (i, 0)),
            pl.BlockSpec((HID, HID), lambda i, k: (0, 0)),
            pl.BlockSpec((1, HID), lambda i, k: (0, 0)),
        ],
        out_specs=ospecs[0] if last else ospecs,
        out_shape=outs[0] if last else outs,
        scratch_shapes=[pltpu.VMEM((RT, YW), jnp.float32)],
    )(s8, y, y, p, x, xsk, wn, bn)
    return res


_BUILD_A = _build_a_kernel()
_SEG128 = _segsum_kernel(YW)


def kernel(x, edge_index, x0, W_in, W_skip, W_conv, b_conv, W_fc, b_fc):
    f32 = jnp.float32
    xp = jnp.pad(x.astype(f32), ((0, NP - N), (0, 0)))
    x0p = jnp.pad(x0.astype(f32), ((0, NP - N), (0, 0)))
    ei = edge_index.astype(jnp.int32)
    src2 = jnp.pad(ei[0], (0, EP - E), constant_values=NP - 1).reshape(ERODS, CH)
    dst2 = jnp.pad(ei[1], (0, EP - E), constant_values=NP - 1).reshape(ERODS, CH)

    xh, xskip, xw1 = _prep(xp, x0p, W_in.astype(f32), W_skip.astype(f32),
                           W_conv[0].astype(f32))
    flat = _flat(src2, dst2)

    a_ref = jax.new_ref(jnp.zeros((NP * NP,), f32))
    _BUILD_A(flat, a_ref)
    a8 = _conv_int8(a_ref[...].reshape(NP, NP))

    b0 = b_conv[0].astype(f32).reshape(1, HID)
    b1 = b_conv[1].astype(f32).reshape(1, HID)
    zb = jnp.zeros((1, HID), f32)
    bfc = b_fc.astype(f32).reshape(1, HID)

    s8 = _a2(a8)
    agg1 = _SEG128(xw1, src2, dst2)
    y1 = _mid(agg1, b0)
    p1 = _SEG128(y1, dst2, src2)
    x1, xw2 = _combine(False, s8, y1, p1, xh, xskip, W_conv[1].astype(f32), zb)

    agg2 = _SEG128(xw2, src2, dst2)
    y2 = _mid(agg2, b1)
    p2 = _SEG128(y2, dst2, src2)
    out = _combine(True, s8, y2, p2, x1, xskip, W_fc.astype(f32), bfc)
    return out[:N]


# speedup vs baseline: 18.8017x; 1.0465x over previous
import functools

import jax
import jax.numpy as jnp
from jax import lax
from jax.experimental import pallas as pl
from jax.experimental.pallas import tpu as pltpu
from jax.experimental.pallas import tpu_sc as plsc

N = 10000
NP = 10240
E = 160000
EP = 163840
NW = 32
NC = 2
NS = 16
CH = 128
NCH = EP // NW // CH
ERODS = EP // CH
YW = 128
HID = 64
RT = 1024
NRT = NP // RT
MT = 1024


def _prep_body(x_ref, x0_ref, win_ref, wskip_ref, wc0_ref,
               xh_ref, xskip_ref, xw1_ref):
    xh = jnp.dot(x_ref[...], win_ref[...], preferred_element_type=jnp.float32)
    x0h = jnp.dot(x0_ref[...], win_ref[...], preferred_element_type=jnp.float32)
    xh_ref[...] = xh
    xskip_ref[...] = jnp.dot(x0h, wskip_ref[...],
                             preferred_element_type=jnp.float32)
    xw1 = jnp.dot(xh, wc0_ref[...], preferred_element_type=jnp.float32)
    xw1_ref[...] = jnp.concatenate(
        [xw1, jnp.zeros((RT, YW - HID), jnp.float32)], axis=1)


def _prep(xp, x0p, w_in, w_skip, wc0):
    inc = xp.shape[1]
    return pl.pallas_call(
        _prep_body,
        grid=(NRT,),
        in_specs=[
            pl.BlockSpec((RT, inc), lambda i: (i, 0)),
            pl.BlockSpec((RT, inc), lambda i: (i, 0)),
            pl.BlockSpec((inc, HID), lambda i: (0, 0)),
            pl.BlockSpec((HID, HID), lambda i: (0, 0)),
            pl.BlockSpec((HID, HID), lambda i: (0, 0)),
        ],
        out_specs=[
            pl.BlockSpec((RT, HID), lambda i: (i, 0)),
            pl.BlockSpec((RT, HID), lambda i: (i, 0)),
            pl.BlockSpec((RT, YW), lambda i: (i, 0)),
        ],
        out_shape=[
            jax.ShapeDtypeStruct((NP, HID), jnp.float32),
            jax.ShapeDtypeStruct((NP, HID), jnp.float32),
            jax.ShapeDtypeStruct((NP, YW), jnp.float32),
        ],
    )(xp, x0p, w_in, w_skip, wc0)


def _flat_body(src_ref, dst_ref, out_ref):
    out_ref[...] = src_ref[...] * NP + dst_ref[...]


def _flat(src2, dst2):
    return pl.pallas_call(
        _flat_body,
        out_shape=jax.ShapeDtypeStruct((ERODS, CH), jnp.int32),
    )(src2, dst2)


def _build_a_kernel():
    mesh = plsc.VectorSubcoreMesh(core_axis_name="c", subcore_axis_name="s")

    grp = 8

    @functools.partial(
        pl.kernel,
        out_type=(),
        mesh=mesh,
        scratch_types=[
            pltpu.VMEM((NCH, CH), jnp.int32),
            pltpu.VMEM((CH,), jnp.float32),
            pltpu.SemaphoreType.DMA,
        ],
    )
    def build_a(flat_hbm, a_ref, idx_v, ones_v, sem):
        c = lax.axis_index("c")
        s = lax.axis_index("s")
        wid = s * NC + c
        for i in range(CH // 16):
            ones_v[pl.ds(i * 16, 16)] = jnp.ones((16,), jnp.float32)
        pltpu.sync_copy(flat_hbm.at[pl.ds(wid * NCH, NCH)], idx_v)

        def scat(g, carry):
            descs = [pltpu.async_copy(ones_v, a_ref.at[idx_v.at[g * grp + u]],
                                      sem) for u in range(grp)]
            for d in descs:
                d.wait()
            return carry

        lax.fori_loop(0, NCH // grp, scat, 0)

    return build_a


def _conv_body(a_ref, o_ref):
    o_ref[...] = a_ref[...].astype(jnp.int4)


def _conv_int8(a2d):
    blk = 256
    return pl.pallas_call(
        _conv_body,
        grid=(NP // blk,),
        in_specs=[pl.BlockSpec((blk, NP), lambda i: (i, 0))],
        out_specs=pl.BlockSpec((blk, NP), lambda i: (i, 0)),
        out_shape=jax.ShapeDtypeStruct((NP, NP), jnp.int4),
    )(a2d)


def _a2_body(a_ref, b_ref, s_ref):
    i = pl.program_id(0)
    j = pl.program_id(1)
    cnt = jnp.dot(a_ref[...], b_ref[...], preferred_element_type=jnp.int32)
    ri = i * MT + lax.broadcasted_iota(jnp.int32, (MT, MT), 0)
    ci = j * MT + lax.broadcasted_iota(jnp.int32, (MT, MT), 1)
    s_ref[...] = ((cnt > 0) & (ri != ci)).astype(jnp.bfloat16)


def _a2(a8):
    g = NP // MT
    return pl.pallas_call(
        _a2_body,
        grid=(g, g),
        in_specs=[
            pl.BlockSpec((MT, NP), lambda i, j: (i, 0)),
            pl.BlockSpec((NP, MT), lambda i, j: (0, j)),
        ],
        out_specs=pl.BlockSpec((MT, MT), lambda i, j: (i, j)),
        out_shape=jax.ShapeDtypeStruct((NP, NP), jnp.bfloat16),
    )(a8, a8)


def _segsum_kernel(W):
    mesh = plsc.VectorSubcoreMesh(core_axis_name="c", subcore_axis_name="s")
    rows_per_tile = NP // NS
    zrows = 32
    nbuf = 2

    @functools.partial(
        pl.kernel,
        out_type=jax.ShapeDtypeStruct((NC, NP, W), jnp.float32),
        mesh=mesh,
        scratch_types=[
            pltpu.VMEM((NCH, CH), jnp.int32),
            pltpu.VMEM((NCH, CH), jnp.int32),
            [pltpu.VMEM((CH, W), jnp.float32) for _ in range(nbuf)],
            pltpu.VMEM((zrows, W), jnp.float32),
            pltpu.VMEM_SHARED((NP, W), jnp.float32),
            [pltpu.SemaphoreType.DMA for _ in range(nbuf)],
        ],
    )
    def seg(tbl_hbm, gidx_hbm, sidx_hbm, out_hbm,
            gidx, sidx, bufs, zbuf, acc, sems):
        c = lax.axis_index("c")
        s = lax.axis_index("s")
        wid = s * NC + c
        for r in range(zrows):
            for col in range(W // 16):
                zbuf[r, pl.ds(col * 16, 16)] = jnp.zeros((16,), jnp.float32)
        row0 = s * rows_per_tile

        def zf(i, carry):
            pltpu.sync_copy(zbuf, acc.at[pl.ds(row0 + i * zrows, zrows)])
            return carry

        lax.fori_loop(0, rows_per_tile // zrows, zf, 0)
        pltpu.sync_copy(gidx_hbm.at[pl.ds(wid * NCH, NCH)], gidx)
        pltpu.sync_copy(sidx_hbm.at[pl.ds(wid * NCH, NCH)], sidx)
        plsc.subcore_barrier()

        def step(g, carry):
            descs = [pltpu.async_copy(tbl_hbm.at[gidx.at[g * nbuf + u]],
                                      bufs[u], sems[u]) for u in range(nbuf)]
            for u in range(nbuf):
                descs[u].wait()
                pltpu.sync_copy(bufs[u], acc.at[sidx.at[g * nbuf + u]],
                                add=True)
            return carry

        lax.fori_loop(0, NCH // nbuf, step, 0)
        plsc.subcore_barrier()

        def wo(i, carry):
            r = row0 + i * zrows * 8
            pltpu.sync_copy(acc.at[pl.ds(r, zrows * 8)],
                            out_hbm.at[c, pl.ds(r, zrows * 8)])
            return carry

        lax.fori_loop(0, rows_per_tile // (zrows * 8), wo, 0)

    return seg


def _mid_body(agg_ref, b_ref, y_ref, yhi_ref, ylo_ref):
    i = pl.program_id(0)
    a = agg_ref[0][:, :HID] + agg_ref[1][:, :HID] + b_ref[...]
    x_agg = jnp.maximum(a, 0.0)
    rid = i * RT + lax.broadcasted_iota(jnp.int32, (RT, 1), 0)
    real = (rid < N).astype(jnp.float32)
    x_agg = x_agg * real
    sqn = jnp.sum(x_agg * x_agg, axis=1, keepdims=True)
    y = jnp.concatenate(
        [x_agg, sqn, real, jnp.zeros((RT, YW - HID - 2), jnp.float32)], axis=1)
    y_ref[...] = y
    yhi = y.astype(jnp.bfloat16)
    yhi_ref[...] = yhi
    ylo_ref[...] = (y - yhi.astype(jnp.float32)).astype(jnp.bfloat16)


def _mid(agg, b):
    return pl.pallas_call(
        _mid_body,
        grid=(NRT,),
        in_specs=[
            pl.BlockSpec((NC, RT, YW), lambda i: (0, i, 0)),
            pl.BlockSpec((1, HID), lambda i: (0, 0)),
        ],
        out_specs=[
            pl.BlockSpec((RT, YW), lambda i: (i, 0)),
            pl.BlockSpec((RT, YW), lambda i: (i, 0)),
            pl.BlockSpec((RT, YW), lambda i: (i, 0)),
        ],
        out_shape=[
            jax.ShapeDtypeStruct((NP, YW), jnp.float32),
            jax.ShapeDtypeStruct((NP, YW), jnp.bfloat16),
            jax.ShapeDtypeStruct((NP, YW), jnp.bfloat16),
        ],
    )(agg, b)


SQT = 512


def _squash_body(s_ref, yhi_ref, ylo_ref, u_ref):
    sb = s_ref[...]
    u_ref[...] = (jnp.dot(sb, yhi_ref[...], preferred_element_type=jnp.float32)
                  + jnp.dot(sb, ylo_ref[...],
                            preferred_element_type=jnp.float32))


def _squash(s8, yhi, ylo):
    return pl.pallas_call(
        _squash_body,
        grid=(NP // SQT,),
        in_specs=[
            pl.BlockSpec((SQT, NP), lambda i: (i, 0)),
            pl.BlockSpec((NP, YW), lambda i: (0, 0)),
            pl.BlockSpec((NP, YW), lambda i: (0, 0)),
        ],
        out_specs=pl.BlockSpec((SQT, YW), lambda i: (i, 0)),
        out_shape=jax.ShapeDtypeStruct((NP, YW), jnp.float32),
    )(s8, yhi, ylo)


def _upd_body(last, u_ref, yi_ref, p_ref, x_ref, xsk_ref, wn_ref, bn_ref,
              *outs):
    u = u_ref[...]
    yi = yi_ref[...]
    x_agg = yi[:, :HID]
    sqn = yi[:, HID:HID + 1]
    pm = p_ref[0] + p_ref[1]
    num_s = (pm[:, HID + 1:HID + 2] * sqn + pm[:, HID:HID + 1]
             - 2.0 * jnp.sum(x_agg * pm[:, :HID], axis=1, keepdims=True))
    g_s = jnp.tanh(num_s / (pm[:, HID + 1:HID + 2] + 1e-10))
    num_q = (u[:, HID + 1:HID + 2] * sqn + u[:, HID:HID + 1]
             - 2.0 * jnp.sum(x_agg * u[:, :HID], axis=1, keepdims=True))
    g_q = jnp.tanh(num_q / (u[:, HID + 1:HID + 2] + 1e-10))
    denom = 1.0 + g_s + g_q
    x_new = (x_ref[...] + g_s * x_agg + g_q * xsk_ref[...]) / denom
    proj = jnp.dot(x_new, wn_ref[...],
                   preferred_element_type=jnp.float32) + bn_ref[...]
    if last:
        outs[0][...] = proj
    else:
        outs[0][...] = x_new
        outs[1][...] = jnp.concatenate(
            [proj, jnp.zeros((RT, YW - HID), jnp.float32)], axis=1)


def _combine(last, s8, yhi, ylo, y, p, x, xsk, wn, bn):
    u = _squash(s8, yhi, ylo)
    outs = [jax.ShapeDtypeStruct((NP, HID), jnp.float32)]
    ospecs = [pl.BlockSpec((RT, HID), lambda i: (i, 0))]
    if not last:
        outs.append(jax.ShapeDtypeStruct((NP, YW), jnp.float32))
        ospecs.append(pl.BlockSpec((RT, YW), lambda i: (i, 0)))
    res = pl.pallas_call(
        functools.partial(_upd_body, last),
        grid=(NRT,),
        in_specs=[
            pl.BlockSpec((RT, YW), lambda i: (i, 0)),
            pl.BlockSpec((RT, YW), lambda i: (i, 0)),
            pl.BlockSpec((NC, RT, YW), lambda i: (0, i, 0)),
            pl.BlockSpec((RT, HID), lambda i: (i, 0)),
            pl.BlockSpec((RT, HID), lambda i: (i, 0)),
            pl.BlockSpec((HID, HID), lambda i: (0, 0)),
            pl.BlockSpec((1, HID), lambda i: (0, 0)),
        ],
        out_specs=ospecs[0] if last else ospecs,
        out_shape=outs[0] if last else outs,
    )(u, y, p, x, xsk, wn, bn)
    return res


_BUILD_A = _build_a_kernel()
_SEG128 = _segsum_kernel(YW)


def kernel(x, edge_index, x0, W_in, W_skip, W_conv, b_conv, W_fc, b_fc):
    f32 = jnp.float32
    xp = jnp.pad(x.astype(f32), ((0, NP - N), (0, 0)))
    x0p = jnp.pad(x0.astype(f32), ((0, NP - N), (0, 0)))
    ei = edge_index.astype(jnp.int32)
    src2 = jnp.pad(ei[0], (0, EP - E), constant_values=NP - 1).reshape(ERODS, CH)
    dst2 = jnp.pad(ei[1], (0, EP - E), constant_values=NP - 1).reshape(ERODS, CH)

    xh, xskip, xw1 = _prep(xp, x0p, W_in.astype(f32), W_skip.astype(f32),
                           W_conv[0].astype(f32))
    flat = _flat(src2, dst2)

    a_ref = jax.new_ref(jnp.zeros((NP * NP,), f32))
    _BUILD_A(flat, a_ref)
    a8 = _conv_int8(a_ref[...].reshape(NP, NP))

    b0 = b_conv[0].astype(f32).reshape(1, HID)
    b1 = b_conv[1].astype(f32).reshape(1, HID)
    zb = jnp.zeros((1, HID), f32)
    bfc = b_fc.astype(f32).reshape(1, HID)

    agg1 = _SEG128(xw1, src2, dst2)
    y1, y1hi, y1lo = _mid(agg1, b0)
    p1 = _SEG128(y1, dst2, src2)
    s8 = _a2(a8)
    x1, xw2 = _combine(False, s8, y1hi, y1lo, y1, p1, xh, xskip,
                       W_conv[1].astype(f32), zb)

    agg2 = _SEG128(xw2, src2, dst2)
    y2, y2hi, y2lo = _mid(agg2, b1)
    p2 = _SEG128(y2, dst2, src2)
    out = _combine(True, s8, y2hi, y2lo, y2, p2, x1, xskip,
                   W_fc.astype(f32), bfc)
    return out[:N]
